# baseline (speedup 1.0000x reference)
import functools
import jax
import jax.numpy as jnp
from jax import lax
from jax.experimental import pallas as pl
from jax.experimental.pallas import tpu as pltpu
from jax.experimental.pallas import tpu_sc as plsc

_BATCH = 16384
_DM, _DS, _DD, _DH = 128, 64, 32, 128
_DOUT = _DM + _DS + _DD + _DH
_NC, _NS, _L = 2, 16, 16
_NW = _NC * _NS
_CHUNK = 128
_NCHUNK = _BATCH // (_NW * _CHUNK)

_mesh = plsc.VectorSubcoreMesh(core_axis_name="c", subcore_axis_name="s")

_ROWBUFS = [
    pltpu.VMEM((_CHUNK, _DM), jnp.float32),
    pltpu.VMEM((_CHUNK, _DS), jnp.float32),
    pltpu.VMEM((_CHUNK, _DD), jnp.float32),
    pltpu.VMEM((_CHUNK, _DH), jnp.float32),
]


@functools.partial(
    pl.kernel,
    mesh=_mesh,
    compiler_params=pltpu.CompilerParams(use_tc_tiling_on_sc=False),
    out_type=jax.ShapeDtypeStruct((_BATCH, _DOUT), jnp.float32),
    scratch_types=[
        pltpu.VMEM((_NCHUNK, _CHUNK), jnp.int32),
        pltpu.VMEM((_NCHUNK, _CHUNK), jnp.int32),
        pltpu.VMEM((_NCHUNK, _CHUNK), jnp.int32),
        pltpu.VMEM((_NCHUNK, _CHUNK), jnp.int32),
        _ROWBUFS,
        _ROWBUFS,
        pltpu.SemaphoreType.DMA,
        pltpu.SemaphoreType.DMA,
        pltpu.SemaphoreType.DMA,
        pltpu.SemaphoreType.DMA,
        pltpu.SemaphoreType.DMA,
    ],
)
def _embedder(month_hbm, day_hbm, hh_hbm, wm, ws, wd, wh, out,
              midx, sidx, didx, hidx, bufs_a, bufs_b,
              semg_a, semg_b, semw_a, semw_b, semi):
    wid = lax.axis_index("s") * _NC + lax.axis_index("c")
    tables = (wm, ws, wd, wh)
    widths = (_DM, _DS, _DD, _DH)
    col0 = (0, _DM, _DM + _DS, _DM + _DS + _DD)
    bufs = (bufs_a, bufs_b)
    semg = (semg_a, semg_b)
    semw = (semw_a, semw_b)

    ci = pltpu.async_copy(month_hbm.at[wid], midx, semi)
    cd = pltpu.async_copy(day_hbm.at[wid], didx, semi)
    ch = pltpu.async_copy(hh_hbm.at[wid], hidx, semi)
    ci.wait()
    cd.wait()
    ch.wait()

    one = jnp.full((_L,), 1, jnp.int32)
    three = jnp.full((_L,), 3, jnp.int32)
    four = jnp.full((_L,), 4, jnp.int32)
    for k in range(_NCHUNK):
        for j in range(_CHUNK // _L):
            m = midx[k, pl.ds(j * _L, _L)]
            sidx[k, pl.ds(j * _L, _L)] = lax.rem(
                lax.div(lax.add(m, one), three), four)

    idxs = (midx, sidx, didx, hidx)

    def fire_gathers(k, b):
        return [
            pltpu.async_copy(tables[t].at[idxs[t].at[k]], bufs[b][t], semg[b])
            for t in range(4)
        ]

    def fire_writes(k, b):
        off = (wid * _NCHUNK + k) * _CHUNK
        return [
            pltpu.async_copy(
                bufs[b][t],
                out.at[pl.ds(off, _CHUNK), pl.ds(col0[t], widths[t])],
                semw[b],
            )
            for t in range(4)
        ]

    def drain(copies):
        for c in copies:
            c.wait()

    g0 = fire_gathers(0, 0)
    g1 = fire_gathers(1, 1)
    drain(g0)
    w0 = fire_writes(0, 0)
    drain(g1)
    w1 = fire_writes(1, 1)
    drain(w0)
    g2 = fire_gathers(2, 0)
    drain(w1)
    g3 = fire_gathers(3, 1)
    drain(g2)
    w2 = fire_writes(2, 0)
    drain(g3)
    w3 = fire_writes(3, 1)
    drain(w2)
    drain(w3)


def kernel(month, day_type, household_id, W_month, W_season, W_day_type, W_household):
    m3 = month.astype(jnp.int32).reshape(_NW, _NCHUNK, _CHUNK)
    d3 = day_type.astype(jnp.int32).reshape(_NW, _NCHUNK, _CHUNK)
    h3 = household_id.astype(jnp.int32).reshape(_NW, _NCHUNK, _CHUNK)
    return _embedder(m3, d3, h3, W_month, W_season, W_day_type, W_household)


# speedup vs baseline: 4.1810x; 4.1810x over previous
import functools
import jax
import jax.numpy as jnp
import numpy as np
from jax import lax
from jax.experimental import pallas as pl
from jax.experimental.pallas import tpu as pltpu
from jax.experimental.pallas import tpu_sc as plsc

_BATCH = 16384
_DM, _DS, _DD, _DH = 128, 64, 32, 128
_DOUT = _DM + _DS + _DD + _DH
_NC, _NS = 2, 16
_NW = _NC * _NS
_CHUNK = 128
_NCHUNK = _BATCH // (_NW * _CHUNK)
_BPW = _NCHUNK * _CHUNK

_M2S = np.zeros((12, 4), np.float32)
_M2S[np.arange(12), [0, 0, 1, 1, 1, 2, 2, 2, 3, 3, 3, 0]] = 1.0

_mesh = plsc.VectorSubcoreMesh(core_axis_name="c", subcore_axis_name="s")


@functools.partial(
    pl.kernel,
    mesh=_mesh,
    compiler_params=pltpu.CompilerParams(use_tc_tiling_on_sc=False),
    out_type=jax.ShapeDtypeStruct((_BATCH, _DH), jnp.float32),
    scratch_types=[
        pltpu.VMEM((_NCHUNK, _CHUNK), jnp.int32),
        pltpu.VMEM((_BPW, _DH), jnp.float32),
        pltpu.SemaphoreType.DMA,
        [pltpu.SemaphoreType.DMA] * _NCHUNK,
        pltpu.SemaphoreType.DMA,
    ],
)
def _hh_gather(hh_hbm, wh, out, hidx, rows, semi, semg, semw):
    wid = lax.axis_index("s") * _NC + lax.axis_index("c")
    pltpu.async_copy(hh_hbm.at[wid], hidx, semi).wait()
    gathers = [
        pltpu.async_copy(
            wh.at[hidx.at[k]],
            rows.at[pl.ds(k * _CHUNK, _CHUNK)],
            semg[k],
        )
        for k in range(_NCHUNK)
    ]
    writes = []
    base = wid * _BPW
    for k in range(_NCHUNK):
        gathers[k].wait()
        writes.append(
            pltpu.async_copy(
                rows.at[pl.ds(k * _CHUNK, _CHUNK)],
                out.at[pl.ds(base + k * _CHUNK, _CHUNK)],
                semw,
            )
        )
    for w in writes:
        w.wait()


def _concat_body(m_ref, d_ref, wm_ref, ws_ref, wd_ref, m2s_ref, hh_ref, o_ref):
    f32 = jnp.float32
    m = m_ref[...]
    d = d_ref[...]
    moh = (m == lax.broadcasted_iota(jnp.int32, (1, 12), 1).astype(f32)).astype(f32)
    doh = (d == lax.broadcasted_iota(jnp.int32, (1, 2), 1).astype(f32)).astype(f32)
    e_m = jnp.dot(moh, wm_ref[...], preferred_element_type=f32)
    soh = jnp.dot(moh, m2s_ref[...], preferred_element_type=f32)
    e_s = jnp.dot(soh, ws_ref[...], preferred_element_type=f32)
    e_d = jnp.dot(doh, wd_ref[...], preferred_element_type=f32)
    o_ref[...] = jnp.concatenate([e_m, e_s, e_d, hh_ref[...]], axis=-1)


_B_BLK = 1024


@jax.jit
def _concat_tc(m_f, d_f, W_month, W_season, W_day_type, hh_rows):
    grid = _BATCH // _B_BLK
    full = lambda shape: pl.BlockSpec(shape, lambda i: (0, 0))
    return pl.pallas_call(
        _concat_body,
        grid=(grid,),
        in_specs=[
            pl.BlockSpec((_B_BLK, 1), lambda i: (i, 0)),
            pl.BlockSpec((_B_BLK, 1), lambda i: (i, 0)),
            full((12, _DM)),
            full((4, _DS)),
            full((2, _DD)),
            full((12, 4)),
            pl.BlockSpec((_B_BLK, _DH), lambda i: (i, 0)),
        ],
        out_specs=pl.BlockSpec((_B_BLK, _DOUT), lambda i: (i, 0)),
        out_shape=jax.ShapeDtypeStruct((_BATCH, _DOUT), jnp.float32),
    )(m_f, d_f, W_month, W_season, W_day_type, jnp.asarray(_M2S), hh_rows)


def kernel(month, day_type, household_id, W_month, W_season, W_day_type, W_household):
    h3 = household_id.astype(jnp.int32).reshape(_NW, _NCHUNK, _CHUNK)
    hh_rows = _hh_gather(h3, W_household)
    m_f = month.astype(jnp.float32).reshape(_BATCH, 1)
    d_f = day_type.astype(jnp.float32).reshape(_BATCH, 1)
    return _concat_tc(m_f, d_f, W_month, W_season, W_day_type, hh_rows)


# speedup vs baseline: 8.3479x; 1.9966x over previous
import functools
import jax
import jax.numpy as jnp
import numpy as np
from jax import lax
from jax.experimental import pallas as pl
from jax.experimental.pallas import tpu as pltpu
from jax.experimental.pallas import tpu_sc as plsc

_BATCH = 16384
_DM, _DS, _DD, _DH = 128, 64, 32, 128
_DOUT = _DM + _DS + _DD + _DH
_NC, _NS = 2, 16
_NW = _NC * _NS
_CHUNK = 128
_NCHUNK = _BATCH // (_NW * _CHUNK)
_BPW = _NCHUNK * _CHUNK

_M2S = np.zeros((12, 4), np.float32)
_M2S[np.arange(12), [0, 0, 1, 1, 1, 2, 2, 2, 3, 3, 3, 0]] = 1.0

_mesh = plsc.VectorSubcoreMesh(core_axis_name="c", subcore_axis_name="s")


@functools.partial(
    pl.kernel,
    mesh=_mesh,
    out_type=jax.ShapeDtypeStruct((_BATCH, _DH), jnp.float32),
    scratch_types=[
        pltpu.VMEM((_NCHUNK, _CHUNK), jnp.int32),
        pltpu.VMEM((_BPW, _DH), jnp.float32),
        pltpu.SemaphoreType.DMA,
        [pltpu.SemaphoreType.DMA] * _NCHUNK,
        pltpu.SemaphoreType.DMA,
    ],
)
def _hh_gather(hh_hbm, wh, out, hidx, rows, semi, semg, semw):
    wid = lax.axis_index("s") * _NC + lax.axis_index("c")
    pltpu.async_copy(hh_hbm.at[wid], hidx, semi).wait()
    gathers = [
        pltpu.async_copy(
            wh.at[hidx.at[k]],
            rows.at[pl.ds(k * _CHUNK, _CHUNK)],
            semg[k],
        )
        for k in range(_NCHUNK)
    ]
    writes = []
    base = wid * _BPW
    for k in range(_NCHUNK):
        gathers[k].wait()
        writes.append(
            pltpu.async_copy(
                rows.at[pl.ds(k * _CHUNK, _CHUNK)],
                out.at[pl.ds(base + k * _CHUNK, _CHUNK)],
                semw,
            )
        )
    for w in writes:
        w.wait()


_B_BLK = 8192


def _concat_body(m_ref, d_ref, wm_ref, ws_ref, wd_ref, m2s_ref, hh_ref, o_ref):
    f32 = jnp.float32
    contract00 = (((0,), (0,)), ((), ()))
    mm = functools.partial(lax.dot_general, dimension_numbers=contract00,
                           preferred_element_type=f32)
    m = m_ref[...].astype(f32)
    d = d_ref[...].astype(f32)
    m2 = lax.broadcast_in_dim(m, (12, _B_BLK), (1,))
    d2 = lax.broadcast_in_dim(d, (2, _B_BLK), (1,))
    moh = (m2 == lax.broadcasted_iota(jnp.int32, (12, _B_BLK), 0).astype(f32)).astype(f32)
    doh = (d2 == lax.broadcasted_iota(jnp.int32, (2, _B_BLK), 0).astype(f32)).astype(f32)
    e_m = mm(wm_ref[...], moh)
    soh = mm(m2s_ref[...], moh)
    e_s = mm(ws_ref[...], soh)
    e_d = mm(wd_ref[...], doh)
    o_ref[: _DM, :] = e_m
    o_ref[_DM : _DM + _DS, :] = e_s
    o_ref[_DM + _DS : _DM + _DS + _DD, :] = e_d
    o_ref[_DM + _DS + _DD :, :] = hh_ref[...].T


def _concat_tc(m_i, d_i, W_month, W_season, W_day_type, hh_rows):
    grid = _BATCH // _B_BLK
    full = lambda shape: pl.BlockSpec(shape, lambda i: (0, 0))
    return pl.pallas_call(
        _concat_body,
        grid=(grid,),
        in_specs=[
            pl.BlockSpec((_B_BLK,), lambda i: (i,)),
            pl.BlockSpec((_B_BLK,), lambda i: (i,)),
            full((12, _DM)),
            full((4, _DS)),
            full((2, _DD)),
            full((12, 4)),
            pl.BlockSpec((_B_BLK, _DH), lambda i: (i, 0)),
        ],
        out_specs=pl.BlockSpec((_DOUT, _B_BLK), lambda i: (0, i)),
        out_shape=jax.ShapeDtypeStruct((_DOUT, _BATCH), jnp.float32),
    )(m_i, d_i, W_month, W_season, W_day_type, jnp.asarray(_M2S), hh_rows)


def kernel(month, day_type, household_id, W_month, W_season, W_day_type, W_household):
    h3 = household_id.astype(jnp.int32).reshape(_NW, _NCHUNK, _CHUNK)
    hh_rows = _hh_gather(h3, W_household)
    out_t = _concat_tc(
        month.astype(jnp.int32), day_type.astype(jnp.int32),
        W_month, W_season, W_day_type, hh_rows)
    return out_t.T
